# Initial kernel scaffold; baseline (speedup 1.0000x reference)
import jax
import jax.numpy as jnp
from jax.experimental import pallas as pl


def kernel(x, edge_index, edge_attr, W, att, bias, bn_gamma, bn_beta):
    raise NotImplementedError("write your pallas kernel here")


# speedup vs baseline: 19.1082x; 19.1082x over previous
import jax
import jax.numpy as jnp
from jax import lax
from jax.experimental import pallas as pl
from jax.experimental.pallas import tpu as pltpu
from jax.experimental.pallas import tpu_sc as plsc

N = 10000
E = 160000
D = 64
H = 4
HD = H * D

_f32 = jnp.float32

NC = 2
NS = 16
NW = NC * NS


def _sp(v):
    return jnp.maximum(v, 0.0) + jnp.log(1.0 + jnp.exp(-jnp.abs(v)))


def _xw_body(x_ref, w_ref, o_ref):
    o_ref[...] = jnp.dot(x_ref[...], w_ref[...], preferred_element_type=_f32)


def _node_matmul(x, W1):
    return pl.pallas_call(
        _xw_body,
        grid=(5,),
        in_specs=[pl.BlockSpec((N // 5, D), lambda i: (i, 0)),
                  pl.BlockSpec((D, HD), lambda i: (0, 0))],
        out_specs=pl.BlockSpec((N // 5, HD), lambda i: (i, 0)),
        out_shape=jax.ShapeDtypeStruct((N, HD), _f32),
    )(x, W1)


BE = 3200
GE = E // BE


def _pass_a_body(gi_ref, gj_ref, ea_ref, w2_ref, ai_ref, aj_ref,
                 ap_ref, st_ref):
    i = pl.program_id(0)
    ea = jnp.dot(ea_ref[...], w2_ref[...], preferred_element_type=_f32)
    ti = _sp(gi_ref[...] + ea)
    tj = _sp(gj_ref[...] + ea)
    s = (jnp.dot(ti, ai_ref[...], preferred_element_type=_f32)
         + jnp.dot(tj, aj_ref[...], preferred_element_type=_f32))
    ap = _sp(s)
    ap_ref[...] = ap
    ssum = jnp.sum(ap, axis=0)
    ssq = jnp.sum(ap * ap, axis=0)
    smx = jnp.max(ap, axis=0)
    smn = jnp.min(ap, axis=0)
    new = jnp.concatenate(
        [ssum[None], ssq[None], smx[None], smn[None],
         jnp.zeros((4, H), _f32)], axis=0)

    @pl.when(i == 0)
    def _():
        st_ref[...] = new

    @pl.when(i > 0)
    def _():
        old = st_ref[...]
        rid = lax.broadcasted_iota(jnp.int32, (8, H), 0)
        st_ref[...] = jnp.where(
            rid < 2, old + new,
            jnp.where(rid == 2, jnp.maximum(old, new),
                      jnp.where(rid == 3, jnp.minimum(old, new), 0.0)))


def _pass_a(gij, edge_attr, W2, Ai, Aj):
    return pl.pallas_call(
        _pass_a_body,
        grid=(GE,),
        in_specs=[pl.BlockSpec((BE, HD), lambda i: (i, 0)),
                  pl.BlockSpec((BE, HD), lambda i: (GE + i, 0)),
                  pl.BlockSpec((BE, D), lambda i: (i, 0)),
                  pl.BlockSpec((D, HD), lambda i: (0, 0)),
                  pl.BlockSpec((HD, H), lambda i: (0, 0)),
                  pl.BlockSpec((HD, H), lambda i: (0, 0))],
        out_specs=[pl.BlockSpec((BE, H), lambda i: (i, 0)),
                   pl.BlockSpec((8, H), lambda i: (0, 0))],
        out_shape=[jax.ShapeDtypeStruct((E, H), _f32),
                   jax.ShapeDtypeStruct((8, H), _f32)],
    )(gij, gij, edge_attr, W2, Ai, Aj)


def _ex_body(ap_ref, p_ref, ex_ref):
    p = p_ref[...]
    mean, rstd, gam, bet, gmax = p[0:1], p[1:2], p[2:3], p[3:4], p[4:5]
    z = gam * (ap_ref[...] - mean) * rstd + bet
    ex_ref[...] = jnp.exp(_sp(z) - gmax)


def _ex_pass(ap, params):
    return pl.pallas_call(
        _ex_body,
        grid=(16,),
        in_specs=[pl.BlockSpec((E // 16, H), lambda i: (i, 0)),
                  pl.BlockSpec((8, H), lambda i: (0, 0))],
        out_specs=pl.BlockSpec((E // 16, H), lambda i: (i, 0)),
        out_shape=jax.ShapeDtypeStruct((E, H), _f32),
    )(ap, params)


def _pass_b_body(gj_ref, ea_ref, w2_ref, ex_ref, dg_ref, eps_ref,
                 em_ref, r_ref, m_ref):
    ea = jnp.dot(ea_ref[...], w2_ref[...], preferred_element_type=_f32)
    tj = _sp(gj_ref[...] + ea)
    alpha = ex_ref[...] / (dg_ref[...] + eps_ref[...])
    aexp = jnp.dot(alpha, em_ref[...], preferred_element_type=_f32)
    m_ref[...] = jnp.dot(tj * aexp, r_ref[...], preferred_element_type=_f32)


def _pass_b(gij, edge_attr, W2, ex, dg, epsv, Emat, R):
    return pl.pallas_call(
        _pass_b_body,
        grid=(GE,),
        in_specs=[pl.BlockSpec((BE, HD), lambda i: (GE + i, 0)),
                  pl.BlockSpec((BE, D), lambda i: (i, 0)),
                  pl.BlockSpec((D, HD), lambda i: (0, 0)),
                  pl.BlockSpec((BE, H), lambda i: (i, 0)),
                  pl.BlockSpec((BE, H), lambda i: (i, 0)),
                  pl.BlockSpec((1, H), lambda i: (0, 0)),
                  pl.BlockSpec((H, HD), lambda i: (0, 0)),
                  pl.BlockSpec((HD, 128), lambda i: (0, 0))],
        out_specs=pl.BlockSpec((BE, 128), lambda i: (i, 0)),
        out_shape=jax.ShapeDtypeStruct((E, 128), _f32),
    )(gij, edge_attr, W2, ex, dg, epsv, Emat, R)


def _ring2(nch, start, finish):
    start(0, 0)
    pairs = (nch - 1) // 2
    if pairs > 0:
        def body(k, _):
            c0 = 2 * k
            start(c0 + 1, 1)
            finish(c0, 0)
            start(c0 + 2, 0)
            finish(c0 + 1, 1)
            return 0
        lax.fori_loop(0, pairs, body, 0)
    if (nch - 1) % 2 == 0:
        finish(nch - 1, 0)
    else:
        start(nch - 1, 1)
        finish(nch - 2, 0)
        finish(nch - 1, 1)


C2 = 80
WT = 2 * E // NW
CH2 = WT // C2


def _gather_body(xw_hbm, ef_hbm, out_hbm, idx_v, r0, r1, s0, s1):
    c = lax.axis_index("c")
    s = lax.axis_index("s")
    base = (c * NS + s) * WT
    pltpu.sync_copy(ef_hbm.at[pl.ds(base, WT)], idx_v)
    bufs = (r0, r1)
    sems = (s0, s1)

    def start(ch, b):
        pltpu.async_copy(xw_hbm.at[idx_v.at[pl.ds(ch * C2, C2)]],
                         bufs[b], sems[b])

    def finish(ch, b):
        pltpu.make_async_copy(xw_hbm.at[idx_v.at[pl.ds(0, C2)]],
                              bufs[b], sems[b]).wait()
        pltpu.sync_copy(bufs[b], out_hbm.at[pl.ds(base + ch * C2, C2)])

    _ring2(CH2, start, finish)


def _edge_gather(xW, eflat):
    f = pl.kernel(
        _gather_body,
        out_type=jax.ShapeDtypeStruct((2 * E, HD), _f32),
        mesh=plsc.VectorSubcoreMesh(core_axis_name="c", subcore_axis_name="s"),
        scratch_types=[
            pltpu.VMEM((WT,), jnp.int32),
            pltpu.VMEM((C2, HD), _f32),
            pltpu.VMEM((C2, HD), _f32),
            pltpu.SemaphoreType.DMA,
            pltpu.SemaphoreType.DMA,
        ],
    )
    return f(xW, eflat)


CS = 40
EW = E // NW
CHS = EW // CS
NP = 10240
NSR = NP // NS


DR = 512
DRT = DR // NS
EG = EW // 4


def _dscat_body(exf_hbm, row_hbm, zr_hbm, parts_hbm, idx_v, ex_v, acc_p):
    c = lax.axis_index("c")
    s = lax.axis_index("s")
    wid = c * NS + s
    pltpu.sync_copy(zr_hbm, acc_p)
    pltpu.sync_copy(row_hbm.at[pl.ds(wid * EW, EW)], idx_v)
    pltpu.sync_copy(exf_hbm.at[pl.ds(wid * 4 * EW, 4 * EW)], ex_v)
    lane = lax.broadcasted_iota(jnp.int32, (16,), 0)
    eoff = lane >> 2
    hoff = lane & 3

    def body(i, _):
        rows4 = plsc.load_gather(idx_v, [i * 4 + eoff])
        vals = ex_v[pl.ds(i * 16, 16)]
        flat = rows4 * H + hoff
        fr = flat >> 7
        fc = flat & 127
        for g in range(4):
            plsc.addupdate_scatter(acc_p, [fr, fc], vals, mask=eoff == g)
        return 0
    lax.fori_loop(0, EG, body, 0)
    pltpu.sync_copy(acc_p, parts_hbm.at[pl.ds(wid * DR, DR)])


def _denom_scatter(exf, row):
    f = pl.kernel(
        _dscat_body,
        out_type=jax.ShapeDtypeStruct((NW * DR, 128), _f32),
        mesh=plsc.VectorSubcoreMesh(core_axis_name="c", subcore_axis_name="s"),
        compiler_params=pltpu.CompilerParams(needs_layout_passes=False),
        scratch_types=[
            pltpu.VMEM((EW,), jnp.int32),
            pltpu.VMEM((4 * EW,), _f32),
            pltpu.VMEM((DR, 128), _f32),
        ],
    )
    return f(exf, row, jnp.zeros((DR, 128), _f32))


def _dgath_body(dn_hbm, row_hbm, out_hbm, idx_v, dn_v, out_v):
    c = lax.axis_index("c")
    s = lax.axis_index("s")
    wid = c * NS + s
    pltpu.sync_copy(dn_hbm, dn_v)
    pltpu.sync_copy(row_hbm.at[pl.ds(wid * EW, EW)], idx_v)
    lane = lax.broadcasted_iota(jnp.int32, (16,), 0)
    eoff = lane >> 2
    hoff = lane & 3

    def body(i, _):
        rows4 = plsc.load_gather(idx_v, [i * 4 + eoff])
        flat = rows4 * H + hoff
        vals = plsc.load_gather(dn_v, [flat >> 7, flat & 127])
        out_v[pl.ds(i * 16, 16)] = vals
        return 0
    lax.fori_loop(0, EG, body, 0)
    pltpu.sync_copy(out_v, out_hbm.at[pl.ds(wid * 4 * EW, 4 * EW)])


def _denom_gather(denomf, row):
    f = pl.kernel(
        _dgath_body,
        out_type=jax.ShapeDtypeStruct((4 * E,), _f32),
        mesh=plsc.VectorSubcoreMesh(core_axis_name="c", subcore_axis_name="s"),
        compiler_params=pltpu.CompilerParams(needs_layout_passes=False),
        scratch_types=[
            pltpu.VMEM((EW,), jnp.int32),
            pltpu.VMEM((DR, 128), _f32),
            pltpu.VMEM((4 * EW,), _f32),
        ],
    )
    return f(denomf, row)


def _scat_body(m_hbm, row2_hbm, z_hbm, parts_hbm, idx_v, v0, v1, s0, s1, acc):
    c = lax.axis_index("c")
    s = lax.axis_index("s")
    wid = c * NS + s
    base = wid * EW
    pltpu.sync_copy(z_hbm.at[pl.ds(s * NSR, NSR)], acc.at[pl.ds(s * NSR, NSR)])
    pltpu.sync_copy(row2_hbm.at[wid], idx_v)
    plsc.subcore_barrier()
    bufs = (v0, v1)
    sems = (s0, s1)

    def start(ch, b):
        pltpu.async_copy(m_hbm.at[pl.ds(base + ch * CS, CS)], bufs[b], sems[b])

    def finish(ch, b):
        pltpu.make_async_copy(m_hbm.at[pl.ds(0, CS)], bufs[b], sems[b]).wait()
        pltpu.sync_copy(bufs[b], acc.at[idx_v.at[ch]], add=True)

    _ring2(CHS, start, finish)
    plsc.subcore_barrier()
    pltpu.sync_copy(acc.at[pl.ds(s * NSR, NSR)],
                    parts_hbm.at[pl.ds(c * NP + s * NSR, NSR)])


def _scatter_sum(vals, row2, width):
    f = pl.kernel(
        _scat_body,
        out_type=jax.ShapeDtypeStruct((NC * NP, width), _f32),
        mesh=plsc.VectorSubcoreMesh(core_axis_name="c", subcore_axis_name="s"),
        scratch_types=[
            pltpu.VMEM((CHS, CS), jnp.int32),
            pltpu.VMEM((CS, width), _f32),
            pltpu.VMEM((CS, width), _f32),
            pltpu.SemaphoreType.DMA,
            pltpu.SemaphoreType.DMA,
            pltpu.VMEM_SHARED((NP, width), _f32),
        ],
    )
    return f(vals, row2, jnp.zeros((NP, width), _f32))


def kernel(x, edge_index, edge_attr, W, att, bias, bn_gamma, bn_beta):
    W1, W2 = W[:D], W[D:]
    att_i = att[0, :, :D]
    att_j = att[0, :, D:]
    eye = jnp.eye(H, dtype=_f32)
    Ai = (att_i[:, :, None] * eye[:, None, :]).reshape(HD, H)
    Aj = (att_j[:, :, None] * eye[:, None, :]).reshape(HD, H)
    Emat = (eye[:, :, None] * jnp.ones((1, 1, D), _f32)).reshape(H, HD)
    R = jnp.pad(jnp.tile(jnp.eye(D, dtype=_f32), (H, 1)), ((0, 0), (0, 128 - D)))

    xW = _node_matmul(x, W1)
    eflat = edge_index.reshape(2 * E)
    gij = _edge_gather(xW, eflat)

    ap, stats = _pass_a(gij, edge_attr, W2, Ai, Aj)
    mean = stats[0] / E
    var = jnp.maximum(stats[1] / E - mean * mean, 0.0)
    rstd = lax.rsqrt(var + 1e-5)
    c1 = _sp(bn_gamma * (stats[2] - mean) * rstd + bn_beta)
    c2 = _sp(bn_gamma * (stats[3] - mean) * rstd + bn_beta)
    gmax = jnp.maximum(c1, c2)
    params = jnp.concatenate(
        [mean[None], rstd[None], bn_gamma[None], bn_beta[None], gmax[None],
         jnp.zeros((3, H), _f32)], axis=0)
    ex = _ex_pass(ap, params)

    row = edge_index[0]
    row2 = row.reshape(NW, CHS, CS)
    exf = ex.reshape(4 * E)
    partsd = _denom_scatter(exf, row)
    denomf = partsd.reshape(NW, DR, 128).sum(axis=0)
    dg = _denom_gather(denomf, row).reshape(E, H)

    epsv = (1e-16 * jnp.exp(-gmax))[None]
    m = _pass_b(gij, edge_attr, W2, ex, dg, epsv, Emat, R)
    parts2 = _scatter_sum(m, row2, 128)
    out = (parts2[:N, :D] + parts2[NP:NP + N, :D]) * (1.0 / H) + bias
    return out


# speedup vs baseline: 23.7398x; 1.2424x over previous
import jax
import jax.numpy as jnp
from jax import lax
from jax.experimental import pallas as pl
from jax.experimental.pallas import tpu as pltpu
from jax.experimental.pallas import tpu_sc as plsc

N = 10000
E = 160000
D = 64
H = 4
HD = H * D

_f32 = jnp.float32

NC = 2
NS = 16
NW = NC * NS


def _sp(v):
    return jnp.maximum(v, 0.0) + jnp.log(1.0 + jnp.exp(-jnp.abs(v)))


def _xw_body(x_ref, w_ref, o_ref):
    o_ref[...] = jnp.dot(x_ref[...], w_ref[...], preferred_element_type=_f32)


def _node_matmul(x, W1):
    return pl.pallas_call(
        _xw_body,
        grid=(5,),
        in_specs=[pl.BlockSpec((N // 5, D), lambda i: (i, 0)),
                  pl.BlockSpec((D, HD), lambda i: (0, 0))],
        out_specs=pl.BlockSpec((N // 5, HD), lambda i: (i, 0)),
        out_shape=jax.ShapeDtypeStruct((N, HD), _f32),
    )(x, W1)


BE = 3200
GE = E // BE


def _pass_a_body(gi_ref, gj_ref, ea_ref, w2_ref, ai_ref, aj_ref,
                 ap_ref, st_ref):
    i = pl.program_id(0)
    ea = jnp.dot(ea_ref[...], w2_ref[...], preferred_element_type=_f32)
    ti = _sp(gi_ref[...] + ea)
    tj = _sp(gj_ref[...] + ea)
    dn = (((0,), (1,)), ((), ()))
    s = (lax.dot_general(ai_ref[...], ti, dn, preferred_element_type=_f32)
         + lax.dot_general(aj_ref[...], tj, dn, preferred_element_type=_f32))
    ap = _sp(s)
    ap_ref[...] = ap
    ssum = jnp.sum(ap, axis=1)
    ssq = jnp.sum(ap * ap, axis=1)
    smx = jnp.max(ap, axis=1)
    smn = jnp.min(ap, axis=1)
    new = jnp.concatenate(
        [ssum[:, None], ssq[:, None], smx[:, None], smn[:, None],
         jnp.zeros((H, 4), _f32)], axis=1)

    @pl.when(i == 0)
    def _():
        st_ref[...] = new

    @pl.when(i > 0)
    def _():
        old = st_ref[...]
        cid = lax.broadcasted_iota(jnp.int32, (H, 8), 1)
        st_ref[...] = jnp.where(
            cid < 2, old + new,
            jnp.where(cid == 2, jnp.maximum(old, new),
                      jnp.where(cid == 3, jnp.minimum(old, new), 0.0)))


def _pass_a(gij, edge_attr, W2, Ai, Aj):
    return pl.pallas_call(
        _pass_a_body,
        grid=(GE,),
        in_specs=[pl.BlockSpec((BE, HD), lambda i: (i, 0)),
                  pl.BlockSpec((BE, HD), lambda i: (GE + i, 0)),
                  pl.BlockSpec((BE, D), lambda i: (i, 0)),
                  pl.BlockSpec((D, HD), lambda i: (0, 0)),
                  pl.BlockSpec((HD, H), lambda i: (0, 0)),
                  pl.BlockSpec((HD, H), lambda i: (0, 0))],
        out_specs=[pl.BlockSpec((H, BE), lambda i: (0, i)),
                   pl.BlockSpec((H, 8), lambda i: (0, 0))],
        out_shape=[jax.ShapeDtypeStruct((H, E), _f32),
                   jax.ShapeDtypeStruct((H, 8), _f32)],
    )(gij, gij, edge_attr, W2, Ai, Aj)


def _ex_body(ap_ref, p_ref, ex_ref):
    p = p_ref[...]
    mean, rstd, gam, bet, gmax = (p[:, 0:1], p[:, 1:2], p[:, 2:3],
                                  p[:, 3:4], p[:, 4:5])
    z = gam * (ap_ref[...] - mean) * rstd + bet
    ex_ref[...] = jnp.exp(_sp(z) - gmax)


def _ex_pass(ap, params):
    return pl.pallas_call(
        _ex_body,
        grid=(10,),
        in_specs=[pl.BlockSpec((H, E // 10), lambda i: (0, i)),
                  pl.BlockSpec((H, 8), lambda i: (0, 0))],
        out_specs=pl.BlockSpec((H, E // 10), lambda i: (0, i)),
        out_shape=jax.ShapeDtypeStruct((H, E), _f32),
    )(ap, params)


def _pass_b_body(gj_ref, ea_ref, w2_ref, ex_ref, dg_ref, eps_ref,
                 em_ref, r_ref, m_ref):
    ea = jnp.dot(ea_ref[...], w2_ref[...], preferred_element_type=_f32)
    tj = _sp(gj_ref[...] + ea)
    alpha = ex_ref[...] / (dg_ref[...] + eps_ref[...])
    dn = (((0,), (0,)), ((), ()))
    aexp = lax.dot_general(alpha, em_ref[...], dn,
                           preferred_element_type=_f32)
    m_ref[...] = jnp.dot(tj * aexp, r_ref[...], preferred_element_type=_f32)


def _pass_b(gij, edge_attr, W2, ex, dg, epsv, Emat, R):
    return pl.pallas_call(
        _pass_b_body,
        grid=(GE,),
        in_specs=[pl.BlockSpec((BE, HD), lambda i: (GE + i, 0)),
                  pl.BlockSpec((BE, D), lambda i: (i, 0)),
                  pl.BlockSpec((D, HD), lambda i: (0, 0)),
                  pl.BlockSpec((H, BE), lambda i: (0, i)),
                  pl.BlockSpec((H, BE), lambda i: (0, i)),
                  pl.BlockSpec((H, 1), lambda i: (0, 0)),
                  pl.BlockSpec((H, HD), lambda i: (0, 0)),
                  pl.BlockSpec((HD, 128), lambda i: (0, 0))],
        out_specs=pl.BlockSpec((BE, 128), lambda i: (i, 0)),
        out_shape=jax.ShapeDtypeStruct((E, 128), _f32),
    )(gij, edge_attr, W2, ex, dg, epsv, Emat, R)


def _ring2(nch, start, finish):
    start(0, 0)
    pairs = (nch - 1) // 2
    if pairs > 0:
        def body(k, _):
            c0 = 2 * k
            start(c0 + 1, 1)
            finish(c0, 0)
            start(c0 + 2, 0)
            finish(c0 + 1, 1)
            return 0
        lax.fori_loop(0, pairs, body, 0)
    if (nch - 1) % 2 == 0:
        finish(nch - 1, 0)
    else:
        start(nch - 1, 1)
        finish(nch - 2, 0)
        finish(nch - 1, 1)


C2 = 80
WT = 2 * E // NW
CH2 = WT // C2


def _gather_body(xw_hbm, ef_hbm, out_hbm, idx_v, r0, r1, s0, s1):
    c = lax.axis_index("c")
    s = lax.axis_index("s")
    base = (c * NS + s) * WT
    pltpu.sync_copy(ef_hbm.at[pl.ds(base, WT)], idx_v)
    bufs = (r0, r1)
    sems = (s0, s1)

    def start(ch, b):
        pltpu.async_copy(xw_hbm.at[idx_v.at[pl.ds(ch * C2, C2)]],
                         bufs[b], sems[b])

    def finish(ch, b):
        pltpu.make_async_copy(xw_hbm.at[idx_v.at[pl.ds(0, C2)]],
                              bufs[b], sems[b]).wait()
        pltpu.sync_copy(bufs[b], out_hbm.at[pl.ds(base + ch * C2, C2)])

    _ring2(CH2, start, finish)


def _edge_gather(xW, eflat):
    f = pl.kernel(
        _gather_body,
        out_type=jax.ShapeDtypeStruct((2 * E, HD), _f32),
        mesh=plsc.VectorSubcoreMesh(core_axis_name="c", subcore_axis_name="s"),
        scratch_types=[
            pltpu.VMEM((WT,), jnp.int32),
            pltpu.VMEM((C2, HD), _f32),
            pltpu.VMEM((C2, HD), _f32),
            pltpu.SemaphoreType.DMA,
            pltpu.SemaphoreType.DMA,
        ],
    )
    return f(xW, eflat)


CS = 40
EW = E // NW
CHS = EW // CS
NP = 10240
NSR = NP // NS


DR = 512
NCH = E // 128


def _worker_chunks(wid):
    return jnp.where(wid < NCH - 39 * NW, 40, 39)


def _dscat_body(ex_hbm, row_hbm, zr_hbm, parts_hbm, idx_b, ex_b, acc_p):
    c = lax.axis_index("c")
    s = lax.axis_index("s")
    wid = c * NS + s
    pltpu.sync_copy(zr_hbm, acc_p)
    lane = lax.broadcasted_iota(jnp.int32, (16,), 0)
    eoff = lane >> 2
    hoff = lane & 3

    def chunk(k, _):
        off = (wid + k * NW) * 128
        pltpu.sync_copy(row_hbm.at[pl.ds(off, 128)], idx_b)
        pltpu.sync_copy(ex_hbm.at[:, pl.ds(off, 128)], ex_b)

        def grp(g, _):
            rows4 = plsc.load_gather(idx_b, [g * 4 + eoff])
            vals = plsc.load_gather(ex_b, [hoff, g * 4 + eoff])
            flat = rows4 * H + hoff
            fr = flat >> 7
            fc = flat & 127
            for gg in range(4):
                plsc.addupdate_scatter(acc_p, [fr, fc], vals, mask=eoff == gg)
            return 0
        lax.fori_loop(0, 32, grp, 0)
        return 0
    lax.fori_loop(0, _worker_chunks(wid), chunk, 0)
    pltpu.sync_copy(acc_p, parts_hbm.at[pl.ds(wid * DR, DR)])


def _denom_scatter(exT, row):
    f = pl.kernel(
        _dscat_body,
        out_type=jax.ShapeDtypeStruct((NW * DR, 128), _f32),
        mesh=plsc.VectorSubcoreMesh(core_axis_name="c", subcore_axis_name="s"),
        compiler_params=pltpu.CompilerParams(needs_layout_passes=False),
        scratch_types=[
            pltpu.VMEM((128,), jnp.int32),
            pltpu.VMEM((H, 128), _f32),
            pltpu.VMEM((DR, 128), _f32),
        ],
    )
    return f(exT, row, jnp.zeros((DR, 128), _f32))


def _dgath_body(dn_hbm, row_hbm, out_hbm, idx_b, dn_v, out_b):
    c = lax.axis_index("c")
    s = lax.axis_index("s")
    wid = c * NS + s
    pltpu.sync_copy(dn_hbm, dn_v)
    lane = lax.broadcasted_iota(jnp.int32, (16,), 0)
    eoff = lane >> 2
    hoff = lane & 3

    def chunk(k, _):
        off = (wid + k * NW) * 128
        pltpu.sync_copy(row_hbm.at[pl.ds(off, 128)], idx_b)

        def grp(g, _):
            rows4 = plsc.load_gather(idx_b, [g * 4 + eoff])
            flat = rows4 * H + hoff
            vals = plsc.load_gather(dn_v, [flat >> 7, flat & 127])
            plsc.store_scatter(out_b, [hoff, g * 4 + eoff], vals)
            return 0
        lax.fori_loop(0, 32, grp, 0)
        pltpu.sync_copy(out_b, out_hbm.at[:, pl.ds(off, 128)])
        return 0
    lax.fori_loop(0, _worker_chunks(wid), chunk, 0)


def _denom_gather(denomf, row):
    f = pl.kernel(
        _dgath_body,
        out_type=jax.ShapeDtypeStruct((H, E), _f32),
        mesh=plsc.VectorSubcoreMesh(core_axis_name="c", subcore_axis_name="s"),
        compiler_params=pltpu.CompilerParams(needs_layout_passes=False),
        scratch_types=[
            pltpu.VMEM((128,), jnp.int32),
            pltpu.VMEM((DR, 128), _f32),
            pltpu.VMEM((H, 128), _f32),
        ],
    )
    return f(denomf, row)


def _scat_body(m_hbm, row2_hbm, z_hbm, parts_hbm, idx_v, v0, v1, s0, s1, acc):
    c = lax.axis_index("c")
    s = lax.axis_index("s")
    wid = c * NS + s
    base = wid * EW
    pltpu.sync_copy(z_hbm.at[pl.ds(s * NSR, NSR)], acc.at[pl.ds(s * NSR, NSR)])
    pltpu.sync_copy(row2_hbm.at[wid], idx_v)
    plsc.subcore_barrier()
    bufs = (v0, v1)
    sems = (s0, s1)

    def start(ch, b):
        pltpu.async_copy(m_hbm.at[pl.ds(base + ch * CS, CS)], bufs[b], sems[b])

    def finish(ch, b):
        pltpu.make_async_copy(m_hbm.at[pl.ds(0, CS)], bufs[b], sems[b]).wait()
        pltpu.sync_copy(bufs[b], acc.at[idx_v.at[ch]], add=True)

    _ring2(CHS, start, finish)
    plsc.subcore_barrier()
    pltpu.sync_copy(acc.at[pl.ds(s * NSR, NSR)],
                    parts_hbm.at[pl.ds(c * NP + s * NSR, NSR)])


def _scatter_sum(vals, row2, width):
    f = pl.kernel(
        _scat_body,
        out_type=jax.ShapeDtypeStruct((NC * NP, width), _f32),
        mesh=plsc.VectorSubcoreMesh(core_axis_name="c", subcore_axis_name="s"),
        scratch_types=[
            pltpu.VMEM((CHS, CS), jnp.int32),
            pltpu.VMEM((CS, width), _f32),
            pltpu.VMEM((CS, width), _f32),
            pltpu.SemaphoreType.DMA,
            pltpu.SemaphoreType.DMA,
            pltpu.VMEM_SHARED((NP, width), _f32),
        ],
    )
    return f(vals, row2, jnp.zeros((NP, width), _f32))


def kernel(x, edge_index, edge_attr, W, att, bias, bn_gamma, bn_beta):
    W1, W2 = W[:D], W[D:]
    att_i = att[0, :, :D]
    att_j = att[0, :, D:]
    eye = jnp.eye(H, dtype=_f32)
    Ai = (att_i[:, :, None] * eye[:, None, :]).reshape(HD, H)
    Aj = (att_j[:, :, None] * eye[:, None, :]).reshape(HD, H)
    Emat = (eye[:, :, None] * jnp.ones((1, 1, D), _f32)).reshape(H, HD)
    R = jnp.pad(jnp.tile(jnp.eye(D, dtype=_f32), (H, 1)), ((0, 0), (0, 128 - D)))

    xW = _node_matmul(x, W1)
    eflat = edge_index.reshape(2 * E)
    gij = _edge_gather(xW, eflat)

    ap, stats = _pass_a(gij, edge_attr, W2, Ai, Aj)
    mean = stats[:, 0] / E
    var = jnp.maximum(stats[:, 1] / E - mean * mean, 0.0)
    rstd = lax.rsqrt(var + 1e-5)
    c1 = _sp(bn_gamma * (stats[:, 2] - mean) * rstd + bn_beta)
    c2 = _sp(bn_gamma * (stats[:, 3] - mean) * rstd + bn_beta)
    gmax = jnp.maximum(c1, c2)
    params = jnp.concatenate(
        [mean[:, None], rstd[:, None], bn_gamma[:, None], bn_beta[:, None],
         gmax[:, None], jnp.zeros((H, 3), _f32)], axis=1)
    ex = _ex_pass(ap, params)

    row = edge_index[0]
    row2 = row.reshape(NW, CHS, CS)
    partsd = _denom_scatter(ex, row)
    denomf = partsd.reshape(NW, DR, 128).sum(axis=0)
    dg = _denom_gather(denomf, row)

    epsv = (1e-16 * jnp.exp(-gmax))[:, None]
    m = _pass_b(gij, edge_attr, W2, ex, dg, epsv, Emat, R)
    parts2 = _scatter_sum(m, row2, 128)
    out = (parts2[:N, :D] + parts2[NP:NP + N, :D]) * (1.0 / H) + bias
    return out


# speedup vs baseline: 29.6839x; 1.2504x over previous
import jax
import jax.numpy as jnp
from jax import lax
from jax.experimental import pallas as pl
from jax.experimental.pallas import tpu as pltpu
from jax.experimental.pallas import tpu_sc as plsc

N = 10000
E = 160000
D = 64
H = 4
HD = H * D

_f32 = jnp.float32
_bf16 = jnp.bfloat16

NC = 2
NS = 16
NW = NC * NS


def _sp(v):
    return jnp.maximum(v, 0.0) + jnp.log(1.0 + jnp.exp(-jnp.abs(v)))


def _pack(lo, hi):
    ilo = lax.bitcast_convert_type(lo.astype(_bf16).astype(_f32), jnp.int32)
    ihi = lax.bitcast_convert_type(hi.astype(_bf16).astype(_f32), jnp.int32)
    return ihi | lax.shift_right_logical(ilo, 16)


def _unpack(v):
    lo = lax.bitcast_convert_type(lax.shift_left(v, 16), _f32)
    hi = lax.bitcast_convert_type(v & jnp.int32(-65536), _f32)
    return lo, hi


def _xw_body(x_ref, w_ref, o_ref):
    xw = jnp.dot(x_ref[...], w_ref[...], preferred_element_type=_f32)
    o_ref[...] = _pack(xw[:, :128], xw[:, 128:])


def _node_matmul(x, W1):
    return pl.pallas_call(
        _xw_body,
        grid=(5,),
        in_specs=[pl.BlockSpec((N // 5, D), lambda i: (i, 0)),
                  pl.BlockSpec((D, HD), lambda i: (0, 0))],
        out_specs=pl.BlockSpec((N // 5, 128), lambda i: (i, 0)),
        out_shape=jax.ShapeDtypeStruct((N, 128), jnp.int32),
    )(x, W1)


BE = 3200
GE = E // BE


def _pass_a_body(gi_ref, gj_ref, ea_ref, w2_ref,
                 ail_ref, aih_ref, ajl_ref, ajh_ref, ap_ref, st_ref):
    i = pl.program_id(0)
    ea = jnp.dot(ea_ref[...], w2_ref[...], preferred_element_type=_f32)
    gil, gih = _unpack(gi_ref[...])
    gjl, gjh = _unpack(gj_ref[...])
    til = _sp(gil + ea[:, :128])
    tih = _sp(gih + ea[:, 128:])
    tjl = _sp(gjl + ea[:, :128])
    tjh = _sp(gjh + ea[:, 128:])
    dn = (((0,), (1,)), ((), ()))
    s = (lax.dot_general(ail_ref[...], til, dn, preferred_element_type=_f32)
         + lax.dot_general(aih_ref[...], tih, dn, preferred_element_type=_f32)
         + lax.dot_general(ajl_ref[...], tjl, dn, preferred_element_type=_f32)
         + lax.dot_general(ajh_ref[...], tjh, dn, preferred_element_type=_f32))
    ap = _sp(s)
    ap_ref[...] = ap
    ssum = jnp.sum(ap, axis=1)
    ssq = jnp.sum(ap * ap, axis=1)
    smx = jnp.max(ap, axis=1)
    smn = jnp.min(ap, axis=1)
    new = jnp.concatenate(
        [ssum[:, None], ssq[:, None], smx[:, None], smn[:, None],
         jnp.zeros((H, 4), _f32)], axis=1)

    @pl.when(i == 0)
    def _():
        st_ref[...] = new

    @pl.when(i > 0)
    def _():
        old = st_ref[...]
        cid = lax.broadcasted_iota(jnp.int32, (H, 8), 1)
        st_ref[...] = jnp.where(
            cid < 2, old + new,
            jnp.where(cid == 2, jnp.maximum(old, new),
                      jnp.where(cid == 3, jnp.minimum(old, new), 0.0)))


def _pass_a(gp, edge_attr, W2, Ail, Aih, Ajl, Ajh):
    return pl.pallas_call(
        _pass_a_body,
        grid=(GE,),
        in_specs=[pl.BlockSpec((BE, 128), lambda i: (i, 0)),
                  pl.BlockSpec((BE, 128), lambda i: (GE + i, 0)),
                  pl.BlockSpec((BE, D), lambda i: (i, 0)),
                  pl.BlockSpec((D, HD), lambda i: (0, 0)),
                  pl.BlockSpec((128, H), lambda i: (0, 0)),
                  pl.BlockSpec((128, H), lambda i: (0, 0)),
                  pl.BlockSpec((128, H), lambda i: (0, 0)),
                  pl.BlockSpec((128, H), lambda i: (0, 0))],
        out_specs=[pl.BlockSpec((H, BE), lambda i: (0, i)),
                   pl.BlockSpec((H, 8), lambda i: (0, 0))],
        out_shape=[jax.ShapeDtypeStruct((H, E), _f32),
                   jax.ShapeDtypeStruct((H, 8), _f32)],
    )(gp, gp, edge_attr, W2, Ail, Aih, Ajl, Ajh)


def _ex_body(ap_ref, p_ref, ex_ref):
    p = p_ref[...]
    mean, rstd, gam, bet, gmax = (p[:, 0:1], p[:, 1:2], p[:, 2:3],
                                  p[:, 3:4], p[:, 4:5])
    z = gam * (ap_ref[...] - mean) * rstd + bet
    ex_ref[...] = jnp.exp(_sp(z) - gmax)


def _ex_pass(ap, params):
    return pl.pallas_call(
        _ex_body,
        grid=(10,),
        in_specs=[pl.BlockSpec((H, E // 10), lambda i: (0, i)),
                  pl.BlockSpec((H, 8), lambda i: (0, 0))],
        out_specs=pl.BlockSpec((H, E // 10), lambda i: (0, i)),
        out_shape=jax.ShapeDtypeStruct((H, E), _f32),
    )(ap, params)


def _pass_b_body(gj_ref, ea_ref, w2_ref, ex_ref, dg_ref, eps_ref,
                 em_ref, rl_ref, rh_ref, m_ref):
    ea = jnp.dot(ea_ref[...], w2_ref[...], preferred_element_type=_f32)
    gjl, gjh = _unpack(gj_ref[...])
    tjl = _sp(gjl + ea[:, :128])
    tjh = _sp(gjh + ea[:, 128:])
    alpha = ex_ref[...] / (dg_ref[...] + eps_ref[...])
    dn = (((0,), (0,)), ((), ()))
    aexp = lax.dot_general(alpha, em_ref[...], dn,
                           preferred_element_type=_f32)
    m_ref[...] = (
        jnp.dot(tjl * aexp[:, :128], rl_ref[...], preferred_element_type=_f32)
        + jnp.dot(tjh * aexp[:, 128:], rh_ref[...],
                  preferred_element_type=_f32))


def _pass_b(gp, edge_attr, W2, ex, dg, epsv, Emat, Rl, Rh):
    return pl.pallas_call(
        _pass_b_body,
        grid=(GE,),
        in_specs=[pl.BlockSpec((BE, 128), lambda i: (GE + i, 0)),
                  pl.BlockSpec((BE, D), lambda i: (i, 0)),
                  pl.BlockSpec((D, HD), lambda i: (0, 0)),
                  pl.BlockSpec((H, BE), lambda i: (0, i)),
                  pl.BlockSpec((H, BE), lambda i: (0, i)),
                  pl.BlockSpec((H, 1), lambda i: (0, 0)),
                  pl.BlockSpec((H, HD), lambda i: (0, 0)),
                  pl.BlockSpec((128, 128), lambda i: (0, 0)),
                  pl.BlockSpec((128, 128), lambda i: (0, 0))],
        out_specs=pl.BlockSpec((BE, 128), lambda i: (i, 0)),
        out_shape=jax.ShapeDtypeStruct((E, 128), _f32),
    )(gp, edge_attr, W2, ex, dg, epsv, Emat, Rl, Rh)


def _ring2(nch, start, finish):
    start(0, 0)
    pairs = (nch - 1) // 2
    if pairs > 0:
        def body(k, _):
            c0 = 2 * k
            start(c0 + 1, 1)
            finish(c0, 0)
            start(c0 + 2, 0)
            finish(c0 + 1, 1)
            return 0
        lax.fori_loop(0, pairs, body, 0)
    if (nch - 1) % 2 == 0:
        finish(nch - 1, 0)
    else:
        start(nch - 1, 1)
        finish(nch - 2, 0)
        finish(nch - 1, 1)


C2 = 80
WT = 2 * E // NW
CH2 = WT // C2


def _gather_body(xw_hbm, ef_hbm, out_hbm, idx_v, r0, r1, s0, s1):
    c = lax.axis_index("c")
    s = lax.axis_index("s")
    base = (c * NS + s) * WT
    pltpu.sync_copy(ef_hbm.at[pl.ds(base, WT)], idx_v)
    bufs = (r0, r1)
    sems = (s0, s1)

    def start(ch, b):
        pltpu.async_copy(xw_hbm.at[idx_v.at[pl.ds(ch * C2, C2)]],
                         bufs[b], sems[b])

    def finish(ch, b):
        pltpu.make_async_copy(xw_hbm.at[idx_v.at[pl.ds(0, C2)]],
                              bufs[b], sems[b]).wait()
        pltpu.sync_copy(bufs[b], out_hbm.at[pl.ds(base + ch * C2, C2)])

    _ring2(CH2, start, finish)


def _edge_gather(xp, eflat):
    f = pl.kernel(
        _gather_body,
        out_type=jax.ShapeDtypeStruct((2 * E, 128), jnp.int32),
        mesh=plsc.VectorSubcoreMesh(core_axis_name="c", subcore_axis_name="s"),
        scratch_types=[
            pltpu.VMEM((WT,), jnp.int32),
            pltpu.VMEM((C2, 128), jnp.int32),
            pltpu.VMEM((C2, 128), jnp.int32),
            pltpu.SemaphoreType.DMA,
            pltpu.SemaphoreType.DMA,
        ],
    )
    return f(xp, eflat)


CS = 40
EW = E // NW
CHS = EW // CS
NP = 10240
NSR = NP // NS

DR = 512
NCH = E // 128
WCH = 40
EPAD = E + 128


def _dscat_body(ex_hbm, row_hbm, zr_hbm, parts_hbm,
                i0, i1, e0, e1, si0, si1, se0, se1, acc_p):
    c = lax.axis_index("c")
    s = lax.axis_index("s")
    wid = c * NS + s
    pltpu.sync_copy(zr_hbm, acc_p)
    lane = lax.broadcasted_iota(jnp.int32, (16,), 0)
    eoff = lane >> 2
    hoff = lane & 3
    ibufs = (i0, i1)
    ebufs = (e0, e1)
    isems = (si0, si1)
    esems = (se0, se1)

    def start(k, b):
        ch = wid + k * NW
        off = jnp.where(ch < NCH, ch * 128, 0)
        pltpu.async_copy(row_hbm.at[pl.ds(off, 128)], ibufs[b], isems[b])
        pltpu.async_copy(ex_hbm.at[:, pl.ds(off, 128)], ebufs[b], esems[b])

    def finish(k, b):
        ch = wid + k * NW
        valid = ch < NCH
        pltpu.make_async_copy(row_hbm.at[pl.ds(0, 128)], ibufs[b],
                              isems[b]).wait()
        pltpu.make_async_copy(ex_hbm.at[:, pl.ds(0, 128)], ebufs[b],
                              esems[b]).wait()

        def grp(g, _):
            rows4 = plsc.load_gather(ibufs[b], [g * 4 + eoff])
            vals = plsc.load_gather(ebufs[b], [hoff, g * 4 + eoff])
            flat = rows4 * H + hoff
            fr = flat >> 7
            fc = flat & 127
            for gg in range(4):
                plsc.addupdate_scatter(acc_p, [fr, fc], vals,
                                       mask=(eoff == gg) & valid)
            return 0
        lax.fori_loop(0, 32, grp, 0)

    _ring2(WCH, start, finish)
    pltpu.sync_copy(acc_p, parts_hbm.at[pl.ds(wid * DR, DR)])


def _denom_scatter(exT, row):
    f = pl.kernel(
        _dscat_body,
        out_type=jax.ShapeDtypeStruct((NW * DR, 128), _f32),
        mesh=plsc.VectorSubcoreMesh(core_axis_name="c", subcore_axis_name="s"),
        compiler_params=pltpu.CompilerParams(needs_layout_passes=False),
        scratch_types=[
            pltpu.VMEM((128,), jnp.int32),
            pltpu.VMEM((128,), jnp.int32),
            pltpu.VMEM((H, 128), _f32),
            pltpu.VMEM((H, 128), _f32),
            pltpu.SemaphoreType.DMA,
            pltpu.SemaphoreType.DMA,
            pltpu.SemaphoreType.DMA,
            pltpu.SemaphoreType.DMA,
            pltpu.VMEM((DR, 128), _f32),
        ],
    )
    return f(exT, row, jnp.zeros((DR, 128), _f32))


def _dgath_body(dn_hbm, row_hbm, out_hbm, i0, i1, si0, si1, dn_v, out_b):
    c = lax.axis_index("c")
    s = lax.axis_index("s")
    wid = c * NS + s
    pltpu.sync_copy(dn_hbm, dn_v)
    lane = lax.broadcasted_iota(jnp.int32, (16,), 0)
    eoff = lane >> 2
    hoff = lane & 3
    ibufs = (i0, i1)
    isems = (si0, si1)

    def start(k, b):
        ch = wid + k * NW
        off = jnp.where(ch < NCH, ch * 128, 0)
        pltpu.async_copy(row_hbm.at[pl.ds(off, 128)], ibufs[b], isems[b])

    def finish(k, b):
        ch = wid + k * NW
        off = jnp.where(ch < NCH, ch * 128, E)
        pltpu.make_async_copy(row_hbm.at[pl.ds(0, 128)], ibufs[b],
                              isems[b]).wait()

        def grp(g, _):
            rows4 = plsc.load_gather(ibufs[b], [g * 4 + eoff])
            flat = rows4 * H + hoff
            vals = plsc.load_gather(dn_v, [flat >> 7, flat & 127])
            plsc.store_scatter(out_b, [hoff, g * 4 + eoff], vals)
            return 0
        lax.fori_loop(0, 32, grp, 0)
        pltpu.sync_copy(out_b, out_hbm.at[:, pl.ds(off, 128)])

    _ring2(WCH, start, finish)


def _denom_gather(denomf, row):
    f = pl.kernel(
        _dgath_body,
        out_type=jax.ShapeDtypeStruct((H, EPAD), _f32),
        mesh=plsc.VectorSubcoreMesh(core_axis_name="c", subcore_axis_name="s"),
        compiler_params=pltpu.CompilerParams(needs_layout_passes=False),
        scratch_types=[
            pltpu.VMEM((128,), jnp.int32),
            pltpu.VMEM((128,), jnp.int32),
            pltpu.SemaphoreType.DMA,
            pltpu.SemaphoreType.DMA,
            pltpu.VMEM((DR, 128), _f32),
            pltpu.VMEM((H, 128), _f32),
        ],
    )
    return f(denomf, row)


def _scat_body(m_hbm, row2_hbm, z_hbm, parts_hbm, idx_v, v0, v1, s0, s1, acc):
    c = lax.axis_index("c")
    s = lax.axis_index("s")
    wid = c * NS + s
    base = wid * EW
    pltpu.sync_copy(z_hbm.at[pl.ds(s * NSR, NSR)], acc.at[pl.ds(s * NSR, NSR)])
    pltpu.sync_copy(row2_hbm.at[wid], idx_v)
    plsc.subcore_barrier()
    bufs = (v0, v1)
    sems = (s0, s1)

    def start(ch, b):
        pltpu.async_copy(m_hbm.at[pl.ds(base + ch * CS, CS)], bufs[b], sems[b])

    def finish(ch, b):
        pltpu.make_async_copy(m_hbm.at[pl.ds(0, CS)], bufs[b], sems[b]).wait()
        pltpu.sync_copy(bufs[b], acc.at[idx_v.at[ch]], add=True)

    _ring2(CHS, start, finish)
    plsc.subcore_barrier()
    pltpu.sync_copy(acc.at[pl.ds(s * NSR, NSR)],
                    parts_hbm.at[pl.ds(c * NP + s * NSR, NSR)])


def _scatter_sum(vals, row2, width):
    f = pl.kernel(
        _scat_body,
        out_type=jax.ShapeDtypeStruct((NC * NP, width), _f32),
        mesh=plsc.VectorSubcoreMesh(core_axis_name="c", subcore_axis_name="s"),
        scratch_types=[
            pltpu.VMEM((CHS, CS), jnp.int32),
            pltpu.VMEM((CS, width), _f32),
            pltpu.VMEM((CS, width), _f32),
            pltpu.SemaphoreType.DMA,
            pltpu.SemaphoreType.DMA,
            pltpu.VMEM_SHARED((NP, width), _f32),
        ],
    )
    return f(vals, row2, jnp.zeros((NP, width), _f32))


def kernel(x, edge_index, edge_attr, W, att, bias, bn_gamma, bn_beta):
    W1, W2 = W[:D], W[D:]
    att_i = att[0, :, :D]
    att_j = att[0, :, D:]
    eye = jnp.eye(H, dtype=_f32)
    Ai = (att_i[:, :, None] * eye[:, None, :]).reshape(HD, H)
    Aj = (att_j[:, :, None] * eye[:, None, :]).reshape(HD, H)
    Emat = (eye[:, :, None] * jnp.ones((1, 1, D), _f32)).reshape(H, HD)
    R = jnp.pad(jnp.tile(jnp.eye(D, dtype=_f32), (H, 1)),
                ((0, 0), (0, 128 - D)))

    xp = _node_matmul(x, W1)
    eflat = edge_index.reshape(2 * E)
    gp = _edge_gather(xp, eflat)

    ap, stats = _pass_a(gp, edge_attr, W2,
                        Ai[:128], Ai[128:], Aj[:128], Aj[128:])
    mean = stats[:, 0] / E
    var = jnp.maximum(stats[:, 1] / E - mean * mean, 0.0)
    rstd = lax.rsqrt(var + 1e-5)
    c1 = _sp(bn_gamma * (stats[:, 2] - mean) * rstd + bn_beta)
    c2 = _sp(bn_gamma * (stats[:, 3] - mean) * rstd + bn_beta)
    gmax = jnp.maximum(c1, c2)
    params = jnp.concatenate(
        [mean[:, None], rstd[:, None], bn_gamma[:, None], bn_beta[:, None],
         gmax[:, None], jnp.zeros((H, 3), _f32)], axis=1)
    ex = _ex_pass(ap, params)

    row = edge_index[0]
    row2 = row.reshape(NW, CHS, CS)
    partsd = _denom_scatter(ex, row)
    denomf = partsd.reshape(NW, DR, 128).sum(axis=0)
    dg = _denom_gather(denomf, row)

    epsv = (1e-16 * jnp.exp(-gmax))[:, None]
    m = _pass_b(gp, edge_attr, W2, ex, dg, epsv, Emat, R[:128], R[128:])
    parts2 = _scatter_sum(m, row2, 128)
    out = (parts2[:N, :D] + parts2[NP:NP + N, :D]) * (1.0 / H) + bias
    return out
